# Initial kernel scaffold; baseline (speedup 1.0000x reference)
import jax
import jax.numpy as jnp
from jax.experimental import pallas as pl


def kernel(features, adj_matrix, W0, b0, W1, b1, Wd, bd):
    raise NotImplementedError("write your pallas kernel here")


# speedup vs baseline: 1.2692x; 1.2692x over previous
import functools

import jax
import jax.numpy as jnp
from jax.experimental import pallas as pl
from jax.experimental.pallas import tpu as pltpu

TI = 512


def _layer0_body(adj_ref, feat_ref, w_ref, b_ref, out_ref, abf_ref):
    i = pl.program_id(1)
    a_i32 = adj_ref[0]
    a = a_i32.astype(jnp.bfloat16)
    abf_ref[0] = a
    deg = jnp.sum(a_i32, axis=1).astype(jnp.float32)[:, None]
    feat = feat_ref[0]
    agg = jax.lax.dot_general(
        a, feat.astype(jnp.bfloat16),
        (((1,), (0,)), ((), ())),
        preferred_element_type=jnp.float32,
    )
    agg = jnp.where(deg > 0, agg / jnp.maximum(deg, 1.0), 0.0)
    self_rows = feat_ref[0, pl.ds(i * TI, TI), :]
    inp = jnp.concatenate([self_rows, agg], axis=1)
    h = jax.nn.sigmoid(
        jax.lax.dot_general(inp, w_ref[...],
                            (((1,), (1,)), ((), ())),
                            preferred_element_type=jnp.float32)
        + b_ref[...]
    )
    norm = jnp.sqrt(jnp.sum(h * h, axis=1, keepdims=True))
    out_ref[0] = h / (norm + 1e-6)


def _layer1_body(abf_ref, out0_ref, w_ref, b_ref, wd_ref, bd_ref, lab_ref):
    i = pl.program_id(1)
    a = abf_ref[0]
    deg = jnp.sum(a, axis=1, dtype=jnp.float32)[:, None]
    out0 = out0_ref[0]
    agg = jax.lax.dot_general(
        a, out0.astype(jnp.bfloat16),
        (((1,), (0,)), ((), ())),
        preferred_element_type=jnp.float32,
    )
    agg = jnp.where(deg > 0, agg / jnp.maximum(deg, 1.0), 0.0)
    self_rows = out0_ref[0, pl.ds(i * TI, TI), :]
    inp = jnp.concatenate([self_rows, agg], axis=1)
    h = jax.nn.sigmoid(
        jax.lax.dot_general(inp, w_ref[...],
                            (((1,), (1,)), ((), ())),
                            preferred_element_type=jnp.float32)
        + b_ref[...]
    )
    norm = jnp.sqrt(jnp.sum(h * h, axis=1, keepdims=True))
    out1 = h / (norm + 1e-6)
    lab = jax.nn.sigmoid(
        jax.lax.dot_general(out1, wd_ref[...],
                            (((1,), (0,)), ((), ())),
                            preferred_element_type=jnp.float32)
        + bd_ref[...]
    )
    lab_ref[0] = lab


@jax.jit
def kernel(features, adj_matrix, W0, b0, W1, b1, Wd, bd):
    B, n, d = features.shape
    ni = n // TI
    b0r = b0.reshape(1, -1)
    b1r = b1.reshape(1, -1)
    wdt = Wd.reshape(-1, 1)
    bdr = bd.reshape(1, 1)

    out0, a_bf16 = pl.pallas_call(
        _layer0_body,
        grid=(B, ni),
        in_specs=[
            pl.BlockSpec((1, TI, n), lambda b, i: (b, i, 0)),
            pl.BlockSpec((1, n, d), lambda b, i: (b, 0, 0)),
            pl.BlockSpec((d, 2 * d), lambda b, i: (0, 0)),
            pl.BlockSpec((1, d), lambda b, i: (0, 0)),
        ],
        out_specs=[
            pl.BlockSpec((1, TI, d), lambda b, i: (b, i, 0)),
            pl.BlockSpec((1, TI, n), lambda b, i: (b, i, 0)),
        ],
        out_shape=[
            jax.ShapeDtypeStruct((B, n, d), jnp.float32),
            jax.ShapeDtypeStruct((B, n, n), jnp.bfloat16),
        ],
        compiler_params=pltpu.CompilerParams(
            dimension_semantics=("arbitrary", "arbitrary"),
        ),
    )(adj_matrix, features, W0, b0r)

    labels = pl.pallas_call(
        _layer1_body,
        grid=(B, ni),
        in_specs=[
            pl.BlockSpec((1, TI, n), lambda b, i: (b, i, 0)),
            pl.BlockSpec((1, n, d), lambda b, i: (b, 0, 0)),
            pl.BlockSpec((d, 2 * d), lambda b, i: (0, 0)),
            pl.BlockSpec((1, d), lambda b, i: (0, 0)),
            pl.BlockSpec((d, 1), lambda b, i: (0, 0)),
            pl.BlockSpec((1, 1), lambda b, i: (0, 0)),
        ],
        out_specs=pl.BlockSpec((1, TI, 1), lambda b, i: (b, i, 0)),
        out_shape=jax.ShapeDtypeStruct((B, n, 1), jnp.float32),
        compiler_params=pltpu.CompilerParams(
            dimension_semantics=("arbitrary", "arbitrary"),
        ),
    )(a_bf16, out0, W1, b1r, wdt, bdr)

    return labels


# speedup vs baseline: 1.4289x; 1.1258x over previous
import jax
import jax.numpy as jnp
from jax.experimental import pallas as pl
from jax.experimental.pallas import tpu as pltpu

TI = 512


def _layer0_body(adj_ref, feat_ref, w_ref, b_ref,
                 out_ref, outq_ref, a8_ref, deg_ref):
    i = pl.program_id(1)
    a_i32 = adj_ref[0]
    a8_ref[0] = a_i32.astype(jnp.int8)
    deg = jnp.sum(a_i32, axis=1).astype(jnp.float32)[:, None]
    deg_ref[0] = deg
    feat = feat_ref[0]
    agg = jax.lax.dot_general(
        a_i32.astype(jnp.bfloat16), feat.astype(jnp.bfloat16),
        (((1,), (0,)), ((), ())),
        preferred_element_type=jnp.float32,
    )
    agg = jnp.where(deg > 0, agg / jnp.maximum(deg, 1.0), 0.0)
    self_rows = feat_ref[0, pl.ds(i * TI, TI), :]
    inp = jnp.concatenate([self_rows, agg], axis=1)
    h = jax.nn.sigmoid(
        jax.lax.dot_general(inp, w_ref[...],
                            (((1,), (1,)), ((), ())),
                            preferred_element_type=jnp.float32)
        + b_ref[...]
    )
    norm = jnp.sqrt(jnp.sum(h * h, axis=1, keepdims=True))
    out0 = h / (norm + 1e-6)
    out_ref[0] = out0
    outq_ref[0] = jnp.round(out0 * 127.0).astype(jnp.int8)


def _layer1_body(a8_ref, out0_ref, outq_ref, deg_ref,
                 w_ref, b_ref, wd_ref, bd_ref, lab_ref):
    i = pl.program_id(1)
    acc = jax.lax.dot_general(
        a8_ref[0], outq_ref[0],
        (((1,), (0,)), ((), ())),
        preferred_element_type=jnp.int32,
    )
    deg = deg_ref[0]
    agg = acc.astype(jnp.float32) * (1.0 / 127.0)
    agg = jnp.where(deg > 0, agg / jnp.maximum(deg, 1.0), 0.0)
    self_rows = out0_ref[0, pl.ds(i * TI, TI), :]
    inp = jnp.concatenate([self_rows, agg], axis=1)
    h = jax.nn.sigmoid(
        jax.lax.dot_general(inp, w_ref[...],
                            (((1,), (1,)), ((), ())),
                            preferred_element_type=jnp.float32)
        + b_ref[...]
    )
    norm = jnp.sqrt(jnp.sum(h * h, axis=1, keepdims=True))
    out1 = h / (norm + 1e-6)
    lab = jax.nn.sigmoid(
        jax.lax.dot_general(out1, wd_ref[...],
                            (((1,), (0,)), ((), ())),
                            preferred_element_type=jnp.float32)
        + bd_ref[...]
    )
    lab_ref[0] = lab


@jax.jit
def kernel(features, adj_matrix, W0, b0, W1, b1, Wd, bd):
    B, n, d = features.shape
    ni = n // TI
    b0r = b0.reshape(1, -1)
    b1r = b1.reshape(1, -1)
    wdt = Wd.reshape(-1, 1)
    bdr = bd.reshape(1, 1)

    out0, out0q, a_i8, deg = pl.pallas_call(
        _layer0_body,
        grid=(B, ni),
        in_specs=[
            pl.BlockSpec((1, TI, n), lambda b, i: (b, i, 0)),
            pl.BlockSpec((1, n, d), lambda b, i: (b, 0, 0)),
            pl.BlockSpec((d, 2 * d), lambda b, i: (0, 0)),
            pl.BlockSpec((1, d), lambda b, i: (0, 0)),
        ],
        out_specs=[
            pl.BlockSpec((1, TI, d), lambda b, i: (b, i, 0)),
            pl.BlockSpec((1, TI, d), lambda b, i: (b, i, 0)),
            pl.BlockSpec((1, TI, n), lambda b, i: (b, i, 0)),
            pl.BlockSpec((1, TI, 1), lambda b, i: (b, i, 0)),
        ],
        out_shape=[
            jax.ShapeDtypeStruct((B, n, d), jnp.float32),
            jax.ShapeDtypeStruct((B, n, d), jnp.int8),
            jax.ShapeDtypeStruct((B, n, n), jnp.int8),
            jax.ShapeDtypeStruct((B, n, 1), jnp.float32),
        ],
        compiler_params=pltpu.CompilerParams(
            dimension_semantics=("arbitrary", "arbitrary"),
        ),
    )(adj_matrix, features, W0, b0r)

    labels = pl.pallas_call(
        _layer1_body,
        grid=(B, ni),
        in_specs=[
            pl.BlockSpec((1, TI, n), lambda b, i: (b, i, 0)),
            pl.BlockSpec((1, n, d), lambda b, i: (b, 0, 0)),
            pl.BlockSpec((1, n, d), lambda b, i: (b, 0, 0)),
            pl.BlockSpec((1, TI, 1), lambda b, i: (b, i, 0)),
            pl.BlockSpec((d, 2 * d), lambda b, i: (0, 0)),
            pl.BlockSpec((1, d), lambda b, i: (0, 0)),
            pl.BlockSpec((d, 1), lambda b, i: (0, 0)),
            pl.BlockSpec((1, 1), lambda b, i: (0, 0)),
        ],
        out_specs=pl.BlockSpec((1, TI, 1), lambda b, i: (b, i, 0)),
        out_shape=jax.ShapeDtypeStruct((B, n, 1), jnp.float32),
        compiler_params=pltpu.CompilerParams(
            dimension_semantics=("arbitrary", "arbitrary"),
        ),
    )(a_i8, out0, out0q, deg, W1, b1r, wdt, bdr)

    return labels


# speedup vs baseline: 1.6847x; 1.1790x over previous
import jax
import jax.numpy as jnp
from jax.experimental import pallas as pl
from jax.experimental.pallas import tpu as pltpu

TI = 512


def _update(self_rows, agg, deg, w_ref, b_ref):
    agg = jnp.where(deg > 0, agg / jnp.maximum(deg, 1.0), 0.0)
    inp = jnp.concatenate([self_rows, agg], axis=1)
    h = jax.nn.sigmoid(
        jax.lax.dot_general(inp, w_ref[...],
                            (((1,), (1,)), ((), ())),
                            preferred_element_type=jnp.float32)
        + b_ref[...]
    )
    norm = jnp.sqrt(jnp.sum(h * h, axis=1, keepdims=True))
    return h / (norm + 1e-6)


def _body(adj_ref, feat_ref, w0_ref, b0_ref, w1_ref, b1_ref,
          wd_ref, bd_ref, lab_ref,
          a8_ref, out0_ref, out0b_ref, deg_ref):
    p = pl.program_id(1)
    i = pl.program_id(2)
    base = i * TI

    @pl.when(p == 0)
    def _layer0():
        a_i32 = adj_ref[0]
        a8_ref[pl.ds(base, TI), :] = a_i32.astype(jnp.int8)
        deg = jnp.sum(a_i32, axis=1).astype(jnp.float32)[:, None]
        deg_ref[pl.ds(base, TI), :] = deg
        feat = feat_ref[0]
        agg = jax.lax.dot_general(
            a_i32.astype(jnp.bfloat16), feat.astype(jnp.bfloat16),
            (((1,), (0,)), ((), ())),
            preferred_element_type=jnp.float32,
        )
        out0 = _update(feat_ref[0, pl.ds(base, TI), :], agg, deg,
                       w0_ref, b0_ref)
        out0_ref[pl.ds(base, TI), :] = out0
        out0b_ref[pl.ds(base, TI), :] = out0.astype(jnp.bfloat16)

    @pl.when(p == 1)
    def _layer1():
        a = a8_ref[pl.ds(base, TI), :].astype(jnp.bfloat16)
        deg = deg_ref[pl.ds(base, TI), :]
        agg = jax.lax.dot_general(
            a, out0b_ref[...],
            (((1,), (0,)), ((), ())),
            preferred_element_type=jnp.float32,
        )
        out1 = _update(out0_ref[pl.ds(base, TI), :], agg, deg,
                       w1_ref, b1_ref)
        lab_ref[0] = jax.nn.sigmoid(
            jax.lax.dot_general(out1, wd_ref[...],
                                (((1,), (0,)), ((), ())),
                                preferred_element_type=jnp.float32)
            + bd_ref[...]
        )


@jax.jit
def kernel(features, adj_matrix, W0, b0, W1, b1, Wd, bd):
    B, n, d = features.shape
    ni = n // TI
    b0r = b0.reshape(1, -1)
    b1r = b1.reshape(1, -1)
    wdt = Wd.reshape(-1, 1)
    bdr = bd.reshape(1, 1)

    labels = pl.pallas_call(
        _body,
        grid=(B, 2, ni),
        in_specs=[
            pl.BlockSpec((1, TI, n),
                         lambda b, p, i: (b, jnp.where(p == 0, i, ni - 1), 0)),
            pl.BlockSpec((1, n, d), lambda b, p, i: (b, 0, 0)),
            pl.BlockSpec((d, 2 * d), lambda b, p, i: (0, 0)),
            pl.BlockSpec((1, d), lambda b, p, i: (0, 0)),
            pl.BlockSpec((d, 2 * d), lambda b, p, i: (0, 0)),
            pl.BlockSpec((1, d), lambda b, p, i: (0, 0)),
            pl.BlockSpec((d, 1), lambda b, p, i: (0, 0)),
            pl.BlockSpec((1, 1), lambda b, p, i: (0, 0)),
        ],
        out_specs=pl.BlockSpec((1, TI, 1), lambda b, p, i: (b, i, 0)),
        out_shape=jax.ShapeDtypeStruct((B, n, 1), jnp.float32),
        scratch_shapes=[
            pltpu.VMEM((n, n), jnp.int8),
            pltpu.VMEM((n, d), jnp.float32),
            pltpu.VMEM((n, d), jnp.bfloat16),
            pltpu.VMEM((n, 1), jnp.float32),
        ],
        compiler_params=pltpu.CompilerParams(
            dimension_semantics=("arbitrary", "arbitrary", "arbitrary"),
        ),
    )(adj_matrix, features, W0, b0r, W1, b1r, wdt, bdr)

    return labels
